# Initial kernel scaffold; baseline (speedup 1.0000x reference)
import jax
import jax.numpy as jnp
from jax.experimental import pallas as pl


def kernel(x, task_id, gate_w, W1, B1, W2, B2, W3, B3, sw1, sb1, sw2, sb2, sw3, sb3, out_w, out_b):
    raise NotImplementedError("write your pallas kernel here")


# speedup vs baseline: 1.0661x; 1.0661x over previous
import functools
import jax
import jax.numpy as jnp
from jax.experimental import pallas as pl
from jax.experimental.pallas import tpu as pltpu

E = 8
TOPK = 2
N = 2048
D = 1024
I = 1024
SI = 1024
OUT = 1024

TM = 256


def _leaky(v):
    return jnp.where(v >= 0, v, 0.01 * v)


def _dot_nt(a, b):
    return jax.lax.dot_general(a, b, (((1,), (1,)), ((), ())),
                               preferred_element_type=jnp.float32)


def _dense_body(x_ref, gate_ref, w1_ref, b1_ref, w2_ref, b2_ref, w3_ref,
                b3_ref, sw1_ref, sb1_ref, sw2_ref, sb2_ref, sw3_ref, sb3_ref,
                ow_ref, ob_ref, out_ref, acc_ref):
    e = pl.program_id(1)
    x = x_ref[...]

    @pl.when(e == 0)
    def _init():
        s1 = _dot_nt(x, sw1_ref[...]) + sb1_ref[...]
        s3 = _dot_nt(x, sw3_ref[...]) + sb3_ref[...]
        acc_ref[...] = _dot_nt(_leaky(s1) * s3, sw2_ref[...]) + sb2_ref[...]

    scores = _dot_nt(x, gate_ref[...])
    p = jax.nn.softmax(scores, axis=-1)
    i1 = jnp.argmax(p, axis=-1)
    m1 = jnp.max(p, axis=-1)
    cols = jax.lax.broadcasted_iota(jnp.int32, p.shape, 1)
    masked = jnp.where(cols == i1[:, None], -jnp.inf, p)
    i2 = jnp.argmax(masked, axis=-1)
    m2 = jnp.max(masked, axis=-1)
    w_e = jnp.where(i1 == e, m1, 0.0) + jnp.where(i2 == e, m2, 0.0)

    w1 = w1_ref[0]
    w3 = w3_ref[0]
    h1 = _dot_nt(x, w1) + b1_ref[e][None, :]
    h3 = _dot_nt(x, w3) + b3_ref[e][None, :]
    eo = _dot_nt(_leaky(h1) * h3, w2_ref[0]) + b2_ref[e][None, :]
    acc_ref[...] += w_e[:, None] * eo

    @pl.when(e == E - 1)
    def _fin():
        out_ref[...] = _dot_nt(acc_ref[...], ow_ref[...]) + ob_ref[...]


@jax.jit
def _dense_moe(x, gate_w, W1, B1, W2, B2, W3, B3, sw1, sb1, sw2, sb2, sw3,
               sb3, out_w, out_b):
    nt = N // TM
    const2 = lambda t, e: (0, 0)
    grid_spec = pltpu.PrefetchScalarGridSpec(
        num_scalar_prefetch=0,
        grid=(nt, E),
        in_specs=[
            pl.BlockSpec((TM, D), lambda t, e: (t, 0)),
            pl.BlockSpec((E, D), const2),
            pl.BlockSpec((1, I, D), lambda t, e: (e, 0, 0)),
            pl.BlockSpec((E, I), const2),
            pl.BlockSpec((1, D, I), lambda t, e: (e, 0, 0)),
            pl.BlockSpec((E, D), const2),
            pl.BlockSpec((1, I, D), lambda t, e: (e, 0, 0)),
            pl.BlockSpec((E, I), const2),
            pl.BlockSpec((SI, D), const2),
            pl.BlockSpec((1, SI), const2),
            pl.BlockSpec((D, SI), const2),
            pl.BlockSpec((1, D), const2),
            pl.BlockSpec((SI, D), const2),
            pl.BlockSpec((1, SI), const2),
            pl.BlockSpec((OUT, D), const2),
            pl.BlockSpec((1, OUT), const2),
        ],
        out_specs=pl.BlockSpec((TM, OUT), lambda t, e: (t, 0)),
        scratch_shapes=[pltpu.VMEM((TM, D), jnp.float32)],
    )
    return pl.pallas_call(
        _dense_body,
        grid_spec=grid_spec,
        out_shape=jax.ShapeDtypeStruct((N, OUT), jnp.float32),
        compiler_params=pltpu.CompilerParams(
            dimension_semantics=("parallel", "arbitrary")),
    )(x, gate_w, W1, B1, W2, B2, W3, B3, sw1, sb1.reshape(1, SI), sw2,
      sb2.reshape(1, D), sw3, sb3.reshape(1, SI), out_w, out_b.reshape(1, OUT))


def kernel(x, task_id, gate_w, W1, B1, W2, B2, W3, B3, sw1, sb1, sw2, sb2,
           sw3, sb3, out_w, out_b):
    xf = x.reshape(N, D)
    return _dense_moe(xf, gate_w, W1, B1, W2, B2, W3, B3, sw1, sb1, sw2, sb2,
                      sw3, sb3, out_w, out_b)
